# Initial kernel scaffold; baseline (speedup 1.0000x reference)
import jax
import jax.numpy as jnp
from jax.experimental import pallas as pl


def kernel(x, edge_index, batch, mask, graph_id_offset, W1a, b1a, W2a, b2a, W1b, b1b, W2b, b2b, Wl1, bl1, Wl2, bl2, Wa, ba):
    raise NotImplementedError("write your pallas kernel here")


# speedup vs baseline: 4.3377x; 4.3377x over previous
import functools

import jax
import jax.numpy as jnp
from jax import lax
from jax.experimental import pallas as pl
from jax.experimental.pallas import tpu as pltpu
from jax.experimental.pallas import tpu_sc as plsc

N_NODES = 10000
N_EDGES = 320000
F = 128
NG = 100
NPG = 100

NC, NS = 2, 16
EPT = N_EDGES // NS
K = 80
NCHUNK = EPT // K

@functools.cache
def _get_sc_agg():
    mesh = plsc.VectorSubcoreMesh(
        core_axis_name="c", subcore_axis_name="s",
        num_cores=NC, num_subcores=NS)
    return pl.kernel(
        _sc_agg_body,
        out_type=jax.ShapeDtypeStruct((2, N_NODES, F), jnp.float32),
        mesh=mesh,
        scratch_types=[
            pltpu.VMEM((K,), jnp.int32),
            pltpu.VMEM((K,), jnp.int32),
            pltpu.VMEM((K, F), jnp.float32),
            pltpu.VMEM_SHARED((N_NODES, F), jnp.float32),
            pltpu.SemaphoreType.DMA,
        ],
    )


def _sc_agg_body(x_hbm, ei_hbm, zeros_hbm, out_hbm, src_v, dst_v, rows_v,
                 agg_sh, sem):
    c = lax.axis_index("c")
    s = lax.axis_index("s")
    rpt = 624
    tail = N_NODES - NS * rpt

    pltpu.sync_copy(zeros_hbm.at[pl.ds(s * rpt, rpt)],
                    agg_sh.at[pl.ds(s * rpt, rpt)])
    @pl.when(s == 0)
    def _():
        pltpu.sync_copy(zeros_hbm.at[pl.ds(NS * rpt, tail)],
                        agg_sh.at[pl.ds(NS * rpt, tail)])
    plsc.subcore_barrier()

    src_base = c * N_EDGES + s * EPT
    dst_base = (1 - c) * N_EDGES + s * EPT

    def body(j, carry):
        off = j * K
        pltpu.sync_copy(ei_hbm.at[pl.ds(src_base + off, K)], src_v)
        pltpu.sync_copy(ei_hbm.at[pl.ds(dst_base + off, K)], dst_v)
        pltpu.async_copy(x_hbm.at[src_v], rows_v, sem).wait()
        pltpu.sync_copy(rows_v, agg_sh.at[dst_v], add=True)
        return carry

    lax.fori_loop(0, NCHUNK, body, 0)
    plsc.subcore_barrier()
    pltpu.sync_copy(agg_sh.at[pl.ds(s * rpt, rpt)],
                    out_hbm.at[c, pl.ds(s * rpt, rpt)])
    @pl.when(s == 0)
    def _():
        pltpu.sync_copy(agg_sh.at[pl.ds(NS * rpt, tail)],
                        out_hbm.at[c, pl.ds(NS * rpt, tail)])


GB = 20
RB = GB * NPG


def _dense_body(x_r, af_r, ar_r, w1a_r, b1a_r, w2a_r, b2a_r,
                w1b_r, b1b_r, w2b_r, b2b_r, wl1_r, bl1_r, wl2_r, bl2_r,
                wa_r, ba_r, out_r):
    xb = x_r[...]

    def mlp(agg, w1, b1, w2, b2):
        h = xb + agg
        h = jnp.maximum(jnp.dot(h, w1, preferred_element_type=jnp.float32)
                        + b1, 0.0)
        return jnp.dot(h, w2, preferred_element_type=jnp.float32) + b2

    hf = mlp(af_r[...], w1a_r[...], b1a_r[...], w2a_r[...], b2a_r[...])
    hr = mlp(ar_r[...], w1b_r[...], b1b_r[...], w2b_r[...], b2b_r[...])

    gi = lax.broadcasted_iota(jnp.int32, (GB, RB), 0)
    ni = lax.broadcasted_iota(jnp.int32, (GB, RB), 1) // NPG
    sel = (gi == ni).astype(jnp.float32)
    ug = lax.broadcasted_iota(jnp.int32, (RB, GB), 0) // NPG
    gg = lax.broadcasted_iota(jnp.int32, (RB, GB), 1)
    unsel = (ug == gg).astype(jnp.float32)

    poolf = jnp.dot(sel, hf, preferred_element_type=jnp.float32) / 100.0
    poolr = jnp.dot(sel, hr, preferred_element_type=jnp.float32) / 100.0

    wl1 = wl1_r[...]
    gproj = (jnp.dot(poolf, wl1[128:256], preferred_element_type=jnp.float32)
             + jnp.dot(poolr, wl1[384:512],
                       preferred_element_type=jnp.float32))
    z = (jnp.dot(hf, wl1[0:128], preferred_element_type=jnp.float32)
         + jnp.dot(hr, wl1[256:384], preferred_element_type=jnp.float32)
         + jnp.dot(unsel, gproj, preferred_element_type=jnp.float32)
         + bl1_r[...])
    z = jnp.maximum(z, 0.0)

    wl2 = wl2_r[...]
    z2 = (jnp.dot(z, wl2[0:128], preferred_element_type=jnp.float32)
          + jnp.dot(xb, wl2[128:256], preferred_element_type=jnp.float32)
          + bl2_r[...])
    z2 = jnp.maximum(z2, 0.0)

    out_r[...] = jnp.dot(z2, wa_r[...],
                         preferred_element_type=jnp.float32) + ba_r[...]


def _full(shape):
    return pl.BlockSpec(shape, lambda i: (0, 0))


_dense = pl.pallas_call(
    _dense_body,
    grid=(N_NODES // RB,),
    in_specs=[
        pl.BlockSpec((RB, F), lambda i: (i, 0)),
        pl.BlockSpec((RB, F), lambda i: (i, 0)),
        pl.BlockSpec((RB, F), lambda i: (i, 0)),
        _full((F, F)), _full((1, F)), _full((F, F)), _full((1, F)),
        _full((F, F)), _full((1, F)), _full((F, F)), _full((1, F)),
        _full((4 * F, F)), _full((1, F)),
        _full((2 * F, F)), _full((1, F)),
        _full((F, 1)), _full((1, 1)),
    ],
    out_specs=pl.BlockSpec((RB, 1), lambda i: (i, 0)),
    out_shape=jax.ShapeDtypeStruct((N_NODES, 1), jnp.float32),
)


def _sample_body(lg_r, pen_r, gum_r, offs_r, samp_r, la_r):
    v = lg_r[...] - pen_r[...]
    m = jnp.max(v, axis=1, keepdims=True)
    e = jnp.exp(v - m)
    probs = e / jnp.sum(e, axis=1, keepdims=True)
    t = jnp.log(probs + 1e-20) + gum_r[...]
    tm = jnp.max(t, axis=1, keepdims=True)
    col = lax.broadcasted_iota(jnp.int32, (NG, NPG), 1)
    idx = jnp.min(jnp.where(t == tm, col, jnp.int32(2**30)), axis=1,
                  keepdims=True)
    p_sel = jnp.sum(jnp.where(col == idx, probs, 0.0), axis=1, keepdims=True)
    samp_r[...] = idx + offs_r[...]
    la_r[...] = jnp.log(p_sel)


_sample = pl.pallas_call(
    _sample_body,
    out_shape=(jax.ShapeDtypeStruct((NG, 1), jnp.int32),
               jax.ShapeDtypeStruct((NG, 1), jnp.float32)),
)


def kernel(x, edge_index, batch, mask, graph_id_offset,
           W1a, b1a, W2a, b2a, W1b, b1b, W2b, b2b,
           Wl1, bl1, Wl2, bl2, Wa, ba):
    zeros = jnp.zeros((N_NODES, F), jnp.float32)
    agg = _get_sc_agg()(x, edge_index.reshape(2 * N_EDGES), zeros)

    logits = _dense(x, agg[0], agg[1],
                    W1a, b1a.reshape(1, F), W2a, b2a.reshape(1, F),
                    W1b, b1b.reshape(1, F), W2b, b2b.reshape(1, F),
                    Wl1, bl1.reshape(1, F), Wl2, bl2.reshape(1, F),
                    Wa, ba.reshape(1, 1))

    lg = logits.reshape(NG, NPG)
    pen = jnp.where(mask, 0.0, 1e10).astype(jnp.float32).reshape(NG, NPG)
    gum = jax.random.gumbel(jax.random.key(42), (NG, NPG), jnp.float32)
    samp, la = _sample(lg, pen, gum, graph_id_offset.reshape(NG, 1))
    return samp.reshape(NG), la.reshape(NG)


# speedup vs baseline: 6.2388x; 1.4383x over previous
import functools

import jax
import jax.numpy as jnp
from jax import lax
from jax.experimental import pallas as pl
from jax.experimental.pallas import tpu as pltpu
from jax.experimental.pallas import tpu_sc as plsc

N_NODES = 10000
N_EDGES = 320000
F = 128
NG = 100
NPG = 100

NC, NS = 2, 16
EPT = N_EDGES // NS
K = 80
NCHUNK = EPT // K

@functools.cache
def _get_sc_agg():
    mesh = plsc.VectorSubcoreMesh(
        core_axis_name="c", subcore_axis_name="s",
        num_cores=NC, num_subcores=NS)
    return pl.kernel(
        _sc_agg_body,
        out_type=jax.ShapeDtypeStruct((2, N_NODES, F), jnp.float32),
        mesh=mesh,
        scratch_types=[
            pltpu.VMEM((K,), jnp.int32),
            pltpu.VMEM((K,), jnp.int32),
            pltpu.VMEM((K, F), jnp.float32),
            pltpu.VMEM((K,), jnp.int32),
            pltpu.VMEM((K,), jnp.int32),
            pltpu.VMEM((K, F), jnp.float32),
            pltpu.VMEM_SHARED((N_NODES, F), jnp.float32),
            pltpu.SemaphoreType.DMA,
            pltpu.SemaphoreType.DMA,
            pltpu.SemaphoreType.DMA,
            pltpu.SemaphoreType.DMA,
            pltpu.SemaphoreType.DMA,
            pltpu.SemaphoreType.DMA,
        ],
    )


def _sc_agg_body(x_hbm, ei_hbm, zeros_hbm, out_hbm,
                 src_a, dst_a, rows_a, src_b, dst_b, rows_b, agg_sh,
                 isem_a, gsem_a, ssem_a, isem_b, gsem_b, ssem_b):
    c = lax.axis_index("c")
    s = lax.axis_index("s")
    rpt = 624
    tail = N_NODES - NS * rpt

    pltpu.sync_copy(zeros_hbm.at[pl.ds(s * rpt, rpt)],
                    agg_sh.at[pl.ds(s * rpt, rpt)])
    @pl.when(s == 0)
    def _():
        pltpu.sync_copy(zeros_hbm.at[pl.ds(NS * rpt, tail)],
                        agg_sh.at[pl.ds(NS * rpt, tail)])
    plsc.subcore_barrier()

    src_base = c * N_EDGES + s * EPT
    dst_base = (1 - c) * N_EDGES + s * EPT

    def idx_issue(j, srcv, dstv, sem):
        off = j * K
        pltpu.async_copy(ei_hbm.at[pl.ds(src_base + off, K)], srcv, sem)
        pltpu.async_copy(ei_hbm.at[pl.ds(dst_base + off, K)], dstv, sem)

    def idx_wait(srcv, dstv, sem):
        pltpu.make_async_copy(ei_hbm.at[pl.ds(src_base, K)], srcv, sem).wait()
        pltpu.make_async_copy(ei_hbm.at[pl.ds(dst_base, K)], dstv, sem).wait()

    def gather_issue(srcv, rows, sem):
        pltpu.async_copy(x_hbm.at[srcv], rows, sem)

    def gather_wait(srcv, rows, sem):
        pltpu.make_async_copy(x_hbm.at[srcv], rows, sem).wait()

    def scat_issue(rows, dstv, sem):
        pltpu.async_copy(rows, agg_sh.at[dstv], sem, add=True)

    def scat_wait(rows, dstv, sem):
        pltpu.make_async_copy(rows, agg_sh.at[dstv], sem).wait()

    idx_issue(0, src_a, dst_a, isem_a)
    idx_wait(src_a, dst_a, isem_a)
    gather_issue(src_a, rows_a, gsem_a)
    idx_issue(1, src_b, dst_b, isem_b)

    def body(g, carry):
        gather_wait(src_a, rows_a, gsem_a)
        scat_issue(rows_a, dst_a, ssem_a)

        @pl.when(g > 0)
        def _():
            scat_wait(rows_b, dst_b, ssem_b)
            idx_issue(2 * g + 1, src_b, dst_b, isem_b)
        idx_wait(src_b, dst_b, isem_b)
        gather_issue(src_b, rows_b, gsem_b)
        gather_wait(src_b, rows_b, gsem_b)
        scat_issue(rows_b, dst_b, ssem_b)

        scat_wait(rows_a, dst_a, ssem_a)

        @pl.when(g < NCHUNK // 2 - 1)
        def _():
            idx_issue(2 * g + 2, src_a, dst_a, isem_a)
            idx_wait(src_a, dst_a, isem_a)
            gather_issue(src_a, rows_a, gsem_a)
        return carry

    lax.fori_loop(0, NCHUNK // 2, body, 0)
    scat_wait(rows_b, dst_b, ssem_b)
    plsc.subcore_barrier()
    pltpu.sync_copy(agg_sh.at[pl.ds(s * rpt, rpt)],
                    out_hbm.at[c, pl.ds(s * rpt, rpt)])
    @pl.when(s == 0)
    def _():
        pltpu.sync_copy(agg_sh.at[pl.ds(NS * rpt, tail)],
                        out_hbm.at[c, pl.ds(NS * rpt, tail)])


GB = 20
RB = GB * NPG


def _dense_body(x_r, af_r, ar_r, w1a_r, b1a_r, w2a_r, b2a_r,
                w1b_r, b1b_r, w2b_r, b2b_r, wl1_r, bl1_r, wl2_r, bl2_r,
                wa_r, ba_r, out_r):
    xb = x_r[...]

    def mlp(agg, w1, b1, w2, b2):
        h = xb + agg
        h = jnp.maximum(jnp.dot(h, w1, preferred_element_type=jnp.float32)
                        + b1, 0.0)
        return jnp.dot(h, w2, preferred_element_type=jnp.float32) + b2

    hf = mlp(af_r[...], w1a_r[...], b1a_r[...], w2a_r[...], b2a_r[...])
    hr = mlp(ar_r[...], w1b_r[...], b1b_r[...], w2b_r[...], b2b_r[...])

    gi = lax.broadcasted_iota(jnp.int32, (GB, RB), 0)
    ni = lax.broadcasted_iota(jnp.int32, (GB, RB), 1) // NPG
    sel = (gi == ni).astype(jnp.float32)
    ug = lax.broadcasted_iota(jnp.int32, (RB, GB), 0) // NPG
    gg = lax.broadcasted_iota(jnp.int32, (RB, GB), 1)
    unsel = (ug == gg).astype(jnp.float32)

    poolf = jnp.dot(sel, hf, preferred_element_type=jnp.float32) / 100.0
    poolr = jnp.dot(sel, hr, preferred_element_type=jnp.float32) / 100.0

    wl1 = wl1_r[...]
    gproj = (jnp.dot(poolf, wl1[128:256], preferred_element_type=jnp.float32)
             + jnp.dot(poolr, wl1[384:512],
                       preferred_element_type=jnp.float32))
    z = (jnp.dot(hf, wl1[0:128], preferred_element_type=jnp.float32)
         + jnp.dot(hr, wl1[256:384], preferred_element_type=jnp.float32)
         + jnp.dot(unsel, gproj, preferred_element_type=jnp.float32)
         + bl1_r[...])
    z = jnp.maximum(z, 0.0)

    wl2 = wl2_r[...]
    z2 = (jnp.dot(z, wl2[0:128], preferred_element_type=jnp.float32)
          + jnp.dot(xb, wl2[128:256], preferred_element_type=jnp.float32)
          + bl2_r[...])
    z2 = jnp.maximum(z2, 0.0)

    out_r[...] = jnp.dot(z2, wa_r[...],
                         preferred_element_type=jnp.float32) + ba_r[...]


def _full(shape):
    return pl.BlockSpec(shape, lambda i: (0, 0))


_dense = pl.pallas_call(
    _dense_body,
    grid=(N_NODES // RB,),
    in_specs=[
        pl.BlockSpec((RB, F), lambda i: (i, 0)),
        pl.BlockSpec((RB, F), lambda i: (i, 0)),
        pl.BlockSpec((RB, F), lambda i: (i, 0)),
        _full((F, F)), _full((1, F)), _full((F, F)), _full((1, F)),
        _full((F, F)), _full((1, F)), _full((F, F)), _full((1, F)),
        _full((4 * F, F)), _full((1, F)),
        _full((2 * F, F)), _full((1, F)),
        _full((F, 1)), _full((1, 1)),
    ],
    out_specs=pl.BlockSpec((RB, 1), lambda i: (i, 0)),
    out_shape=jax.ShapeDtypeStruct((N_NODES, 1), jnp.float32),
)


def _sample_body(lg_r, pen_r, gum_r, offs_r, samp_r, la_r):
    v = lg_r[...] - pen_r[...]
    m = jnp.max(v, axis=1, keepdims=True)
    e = jnp.exp(v - m)
    probs = e / jnp.sum(e, axis=1, keepdims=True)
    t = jnp.log(probs + 1e-20) + gum_r[...]
    tm = jnp.max(t, axis=1, keepdims=True)
    col = lax.broadcasted_iota(jnp.int32, (NG, NPG), 1)
    idx = jnp.min(jnp.where(t == tm, col, jnp.int32(2**30)), axis=1,
                  keepdims=True)
    p_sel = jnp.sum(jnp.where(col == idx, probs, 0.0), axis=1, keepdims=True)
    samp_r[...] = idx + offs_r[...]
    la_r[...] = jnp.log(p_sel)


_sample = pl.pallas_call(
    _sample_body,
    out_shape=(jax.ShapeDtypeStruct((NG, 1), jnp.int32),
               jax.ShapeDtypeStruct((NG, 1), jnp.float32)),
)


def kernel(x, edge_index, batch, mask, graph_id_offset,
           W1a, b1a, W2a, b2a, W1b, b1b, W2b, b2b,
           Wl1, bl1, Wl2, bl2, Wa, ba):
    zeros = jnp.zeros((N_NODES, F), jnp.float32)
    agg = _get_sc_agg()(x, edge_index.reshape(2 * N_EDGES), zeros)

    logits = _dense(x, agg[0], agg[1],
                    W1a, b1a.reshape(1, F), W2a, b2a.reshape(1, F),
                    W1b, b1b.reshape(1, F), W2b, b2b.reshape(1, F),
                    Wl1, bl1.reshape(1, F), Wl2, bl2.reshape(1, F),
                    Wa, ba.reshape(1, 1))

    lg = logits.reshape(NG, NPG)
    pen = jnp.where(mask, 0.0, 1e10).astype(jnp.float32).reshape(NG, NPG)
    gum = jax.random.gumbel(jax.random.key(42), (NG, NPG), jnp.float32)
    samp, la = _sample(lg, pen, gum, graph_id_offset.reshape(NG, 1))
    return samp.reshape(NG), la.reshape(NG)
